# baseline (speedup 1.0000x reference)
import functools

import jax
import jax.numpy as jnp
from jax.experimental import pallas as pl
from jax.experimental.pallas import tpu as pltpu
from jax.experimental.pallas import tpu_sc as plsc

B, N, NBR = 4, 2500, 32
F, G = 128, 64

TILE_N = 128
NT = (N + TILE_N - 1) // TILE_N
GATHER_WINDOW = 128


def _init_body(feat_ref, w_ref, out_ref):
    out_ref[...] = jnp.dot(feat_ref[...], w_ref[...],
                           preferred_element_type=jnp.float32)


def _init_features(features_flat, W_init):
    return pl.pallas_call(
        _init_body,
        out_shape=jax.ShapeDtypeStruct((B * N, F), jnp.float32),
    )(features_flat, W_init)


NWORK = 32
NBUF = 4
LOOK = 2


def _sc_gather(table, idx_flat):
    m = idx_flat.shape[1]
    wpw = m // (GATHER_WINDOW * NWORK)
    mesh = plsc.VectorSubcoreMesh(core_axis_name="c", subcore_axis_name="s")

    @functools.partial(
        pl.kernel,
        out_type=jax.ShapeDtypeStruct((m, F), jnp.float32),
        mesh=mesh,
        scratch_types=[
            pltpu.VMEM((wpw * GATHER_WINDOW,), jnp.int32),
            pltpu.VMEM((NBUF, GATHER_WINDOW, F), jnp.float32),
            pltpu.SemaphoreType.DMA,
            pltpu.SemaphoreType.DMA((NBUF,)),
            pltpu.SemaphoreType.DMA((NBUF,)),
        ],
    )
    def k(table_hbm, idx_hbm, out_hbm, idx_v, bufs, isem, gsem, wsem):
        wid = jax.lax.axis_index("s") * 2 + jax.lax.axis_index("c")
        base = wid * (wpw * GATHER_WINDOW)
        pltpu.async_copy(idx_hbm.at[0, pl.ds(base, wpw * GATHER_WINDOW)],
                         idx_v, isem).wait()
        g, w = {}, {}
        for t in range(wpw + LOOK):
            if t < wpw:
                j = t % NBUF
                if t >= NBUF:
                    w[t - NBUF].wait()
                g[t] = pltpu.async_copy(
                    table_hbm.at[idx_v.at[pl.ds(t * GATHER_WINDOW,
                                                GATHER_WINDOW)]],
                    bufs.at[j], gsem.at[j])
            if t >= LOOK:
                u = t - LOOK
                j = u % NBUF
                g[u].wait()
                w[u] = pltpu.async_copy(
                    bufs.at[j],
                    out_hbm.at[pl.ds(base + u * GATHER_WINDOW,
                                     GATHER_WINDOW)],
                    wsem.at[j])
        for u in range(max(0, wpw - NBUF), wpw):
            w[u].wait()

    return k(table, idx_flat)


def _fused_body(rbf_ref, gath_ref, w1_ref, b1_ref, w2_ref, b2_ref, v_ref,
                wo1_ref, bo1_ref, wo2_ref, bo2_ref, out_ref, attn_ref):
    rbf = rbf_ref[...].reshape(TILE_N * NBR, G).astype(jnp.bfloat16)
    h = jnp.tanh(jnp.dot(rbf, w1_ref[...].astype(jnp.bfloat16),
                         preferred_element_type=jnp.float32) + b1_ref[...])
    filt = jnp.dot(h.astype(jnp.bfloat16), w2_ref[...].astype(jnp.bfloat16),
                   preferred_element_type=jnp.float32) + b2_ref[...]
    conv = gath_ref[...].reshape(TILE_N * NBR, F) * filt
    conv3 = conv.reshape(TILE_N, NBR, F)
    logits = jnp.sum(conv3 * v_ref[...].reshape(1, 1, F), axis=-1,
                     keepdims=True)
    m = jnp.max(logits, axis=1, keepdims=True)
    e = jnp.exp(logits - m)
    attn3 = e / jnp.sum(e, axis=1, keepdims=True)
    attn_ref[...] = attn3.reshape(TILE_N, NBR)
    agg = jnp.sum(conv3 * attn3, axis=1)
    out = jnp.dot(jnp.tanh(jnp.dot(agg.astype(jnp.bfloat16),
                                   wo1_ref[...].astype(jnp.bfloat16),
                                   preferred_element_type=jnp.float32)
                           + bo1_ref[...]).astype(jnp.bfloat16),
                  wo2_ref[...].astype(jnp.bfloat16),
                  preferred_element_type=jnp.float32) + bo2_ref[...]
    out_ref[...] = out


def _fused(rbf_b, gath_b, W1, b1, W2, b2, v_row, Wo1, bo1, Wo2, bo2):
    full = lambda shape: pl.BlockSpec(shape, lambda i: tuple(0 for _ in shape))
    return pl.pallas_call(
        _fused_body,
        grid=(NT,),
        in_specs=[
            pl.BlockSpec((TILE_N, NBR, G), lambda i: (i, 0, 0)),
            pl.BlockSpec((TILE_N, NBR, F), lambda i: (i, 0, 0)),
            full((G, F)), full((1, F)), full((F, F)), full((1, F)),
            full((1, F)),
            full((F, F)), full((1, F)), full((F, F)), full((1, F)),
        ],
        out_specs=[
            pl.BlockSpec((TILE_N, F), lambda i: (i, 0)),
            pl.BlockSpec((TILE_N, NBR), lambda i: (i, 0)),
        ],
        out_shape=[
            jax.ShapeDtypeStruct((N, F), jnp.float32),
            jax.ShapeDtypeStruct((N, NBR), jnp.float32),
        ],
    )(rbf_b, gath_b, W1, b1, W2, b2, v_row, Wo1, bo1, Wo2, bo2)


def kernel(features, rbf_expansion, neighbor_list, W_init, W1, b1, W2, b2,
           nbr_filter, Wo1, bo1, Wo2, bo2):
    init = _init_features(features.reshape(B * N, F), W_init)
    b1r, b2r = b1.reshape(1, F), b2.reshape(1, F)
    bo1r, bo2r = bo1.reshape(1, F), bo2.reshape(1, F)
    v_row = nbr_filter.reshape(1, F)
    m_pad = GATHER_WINDOW * NWORK * (
        (N * NBR + GATHER_WINDOW * NWORK - 1) // (GATHER_WINDOW * NWORK))
    n_pad = m_pad // NBR
    pad = jnp.zeros((m_pad - N * NBR,), jnp.int32)
    gaths = []
    for b in range(B):
        table = jax.lax.slice(init, (b * N, 0), ((b + 1) * N, F))
        idx = jnp.concatenate([neighbor_list[b].reshape(-1), pad])
        gaths.append(_sc_gather(table, idx.reshape(1, m_pad)))
    outs, attns = [], []
    for b in range(B):
        out_b, attn_b = _fused(rbf_expansion[b],
                               gaths[b].reshape(n_pad, NBR, F),
                               W1, b1r, W2, b2r, v_row, Wo1, bo1r, Wo2, bo2r)
        outs.append(out_b)
        attns.append(attn_b)
    return jnp.stack(outs), jnp.stack(attns)


# speedup vs baseline: 1.3314x; 1.3314x over previous
import functools

import jax
import jax.numpy as jnp
from jax.experimental import pallas as pl
from jax.experimental.pallas import tpu as pltpu
from jax.experimental.pallas import tpu_sc as plsc

B, N, NBR = 4, 2500, 32
F, G = 128, 64

TILE_N = 128
NT = (N + TILE_N - 1) // TILE_N
GATHER_WINDOW = 128


def _init_body(feat_ref, w_ref, out_ref):
    out_ref[...] = jnp.dot(feat_ref[...], w_ref[...],
                           preferred_element_type=jnp.float32)


def _init_features(features_flat, W_init):
    return pl.pallas_call(
        _init_body,
        out_shape=jax.ShapeDtypeStruct((B * N, F), jnp.float32),
    )(features_flat, W_init)


def _sc_gather(table, idx_flat):
    m = idx_flat.shape[1]
    mesh = plsc.VectorSubcoreMesh(core_axis_name="c", subcore_axis_name="s")

    @functools.partial(
        pl.kernel,
        out_type=jax.ShapeDtypeStruct((m, F), jnp.float32),
        mesh=mesh,
    )
    def k(table_hbm, idx_hbm, out_hbm):
        def body(i_vmem, o_vmem):
            pltpu.sync_copy(table_hbm.at[i_vmem.at[0]], o_vmem)

        pltpu.emit_pipeline(
            body,
            grid=(m // GATHER_WINDOW,),
            in_specs=[pl.BlockSpec((1, GATHER_WINDOW), lambda i: (0, i))],
            out_specs=[pl.BlockSpec((GATHER_WINDOW, F), lambda i: (i, 0))],
            core_axis_name=("c", "s"),
            dimension_semantics=(pltpu.PARALLEL,),
        )(idx_hbm, out_hbm)

    return k(table, idx_flat)


def _filter_body(rbf_ref, w1_ref, b1_ref, w2_ref, b2_ref, out_ref):
    rbf = rbf_ref[...].reshape(TILE_N * NBR, G).astype(jnp.bfloat16)
    h = jnp.tanh(jnp.dot(rbf, w1_ref[...].astype(jnp.bfloat16),
                         preferred_element_type=jnp.float32) + b1_ref[...])
    filt = jnp.dot(h.astype(jnp.bfloat16), w2_ref[...].astype(jnp.bfloat16),
                   preferred_element_type=jnp.float32) + b2_ref[...]
    out_ref[...] = filt.reshape(1, TILE_N, NBR, F).astype(jnp.bfloat16)


def _filter(rbf, W1, b1, W2, b2):
    full = lambda shape: pl.BlockSpec(shape, lambda b, i: tuple(0 for _ in shape))
    return pl.pallas_call(
        _filter_body,
        grid=(B, NT),
        in_specs=[
            pl.BlockSpec((1, TILE_N, NBR, G), lambda b, i: (b, i, 0, 0)),
            full((G, F)), full((1, F)), full((F, F)), full((1, F)),
        ],
        out_specs=pl.BlockSpec((1, TILE_N, NBR, F), lambda b, i: (b, i, 0, 0)),
        out_shape=jax.ShapeDtypeStruct((B, N, NBR, F), jnp.bfloat16),
    )(rbf, W1, b1, W2, b2)


def _combine_body(gath_ref, filt_ref, v_ref, wo1_ref, bo1_ref, wo2_ref,
                  bo2_ref, out_ref, attn_ref):
    conv3 = (gath_ref[...].reshape(TILE_N, NBR, F)
             * filt_ref[...].reshape(TILE_N, NBR, F).astype(jnp.float32))
    logits = jnp.sum(conv3 * v_ref[...].reshape(1, 1, F), axis=-1,
                     keepdims=True)
    m = jnp.max(logits, axis=1, keepdims=True)
    e = jnp.exp(logits - m)
    attn3 = e / jnp.sum(e, axis=1, keepdims=True)
    attn_ref[...] = attn3.reshape(TILE_N, NBR)
    agg = jnp.sum(conv3 * attn3, axis=1)
    out = jnp.dot(jnp.tanh(jnp.dot(agg.astype(jnp.bfloat16),
                                   wo1_ref[...].astype(jnp.bfloat16),
                                   preferred_element_type=jnp.float32)
                           + bo1_ref[...]).astype(jnp.bfloat16),
                  wo2_ref[...].astype(jnp.bfloat16),
                  preferred_element_type=jnp.float32) + bo2_ref[...]
    out_ref[...] = out


def _combine(gath_b, filt_b, v_row, Wo1, bo1, Wo2, bo2):
    full = lambda shape: pl.BlockSpec(shape, lambda i: tuple(0 for _ in shape))
    return pl.pallas_call(
        _combine_body,
        grid=(NT,),
        in_specs=[
            pl.BlockSpec((TILE_N, NBR, F), lambda i: (i, 0, 0)),
            pl.BlockSpec((TILE_N, NBR, F), lambda i: (i, 0, 0)),
            full((1, F)), full((F, F)), full((1, F)), full((F, F)),
            full((1, F)),
        ],
        out_specs=[
            pl.BlockSpec((TILE_N, F), lambda i: (i, 0)),
            pl.BlockSpec((TILE_N, NBR), lambda i: (i, 0)),
        ],
        out_shape=[
            jax.ShapeDtypeStruct((N, F), jnp.float32),
            jax.ShapeDtypeStruct((N, NBR), jnp.float32),
        ],
    )(gath_b, filt_b, v_row, Wo1, bo1, Wo2, bo2)


def kernel(features, rbf_expansion, neighbor_list, W_init, W1, b1, W2, b2,
           nbr_filter, Wo1, bo1, Wo2, bo2):
    init = _init_features(features.reshape(B * N, F), W_init)
    b1r, b2r = b1.reshape(1, F), b2.reshape(1, F)
    bo1r, bo2r = bo1.reshape(1, F), bo2.reshape(1, F)
    v_row = nbr_filter.reshape(1, F)
    gaths = []
    for b in range(B):
        table = jax.lax.slice(init, (b * N, 0), ((b + 1) * N, F))
        gaths.append(_sc_gather(table, neighbor_list[b].reshape(1, N * NBR)))
    filt = _filter(rbf_expansion, W1, b1r, W2, b2r)
    outs, attns = [], []
    for b in range(B):
        out_b, attn_b = _combine(gaths[b].reshape(N, NBR, F), filt[b],
                                 v_row, Wo1, bo1r, Wo2, bo2r)
        outs.append(out_b)
        attns.append(attn_b)
    return jnp.stack(outs), jnp.stack(attns)


# speedup vs baseline: 1.8505x; 1.3899x over previous
import functools

import jax
import jax.numpy as jnp
from jax.experimental import pallas as pl
from jax.experimental.pallas import tpu as pltpu
from jax.experimental.pallas import tpu_sc as plsc

B, N, NBR = 4, 2500, 32
F, G = 128, 64

TILE_N = 128
NT = (N + TILE_N - 1) // TILE_N
GATHER_WINDOW = 128


def _init_body(feat_ref, w_ref, out_ref):
    out_ref[...] = jnp.dot(feat_ref[...], w_ref[...],
                           preferred_element_type=jnp.float32)


def _init_features(features_flat, W_init):
    return pl.pallas_call(
        _init_body,
        out_shape=jax.ShapeDtypeStruct((B * N, F), jnp.float32),
    )(features_flat, W_init)


def _sc_gather(table, idx_flat):
    m = idx_flat.shape[1]
    mesh = plsc.VectorSubcoreMesh(core_axis_name="c", subcore_axis_name="s")

    @functools.partial(
        pl.kernel,
        out_type=jax.ShapeDtypeStruct((m, F), jnp.float32),
        mesh=mesh,
    )
    def k(table_hbm, idx_hbm, out_hbm):
        def body(i_vmem, o_vmem):
            pltpu.sync_copy(table_hbm.at[i_vmem.at[0]], o_vmem)

        pltpu.emit_pipeline(
            body,
            grid=(m // GATHER_WINDOW,),
            in_specs=[pl.BlockSpec((1, GATHER_WINDOW), lambda i: (0, i))],
            out_specs=[pl.BlockSpec((GATHER_WINDOW, F), lambda i: (i, 0))],
            core_axis_name=("c", "s"),
            dimension_semantics=(pltpu.PARALLEL,),
        )(idx_hbm, out_hbm)

    return k(table, idx_flat)


def _fused_body(rbf_ref, gath_ref, w1_ref, b1_ref, w2_ref, b2_ref, v_ref,
                wo1_ref, bo1_ref, wo2_ref, bo2_ref, out_ref, attn_ref):
    rbf = rbf_ref[...].reshape(TILE_N * NBR, G).astype(jnp.bfloat16)
    h = jnp.tanh(jnp.dot(rbf, w1_ref[...].astype(jnp.bfloat16),
                         preferred_element_type=jnp.float32) + b1_ref[...])
    filt = jnp.dot(h.astype(jnp.bfloat16), w2_ref[...].astype(jnp.bfloat16),
                   preferred_element_type=jnp.float32) + b2_ref[...]
    conv = gath_ref[...].reshape(TILE_N * NBR, F) * filt
    conv3 = conv.reshape(TILE_N, NBR, F)
    logits = jnp.sum(conv3 * v_ref[...].reshape(1, 1, F), axis=-1,
                     keepdims=True)
    m = jnp.max(logits, axis=1, keepdims=True)
    e = jnp.exp(logits - m)
    attn3 = e / jnp.sum(e, axis=1, keepdims=True)
    attn_ref[...] = attn3.reshape(1, TILE_N, NBR)
    agg = jnp.sum(conv3 * attn3, axis=1)
    out = jnp.dot(jnp.tanh(jnp.dot(agg.astype(jnp.bfloat16),
                                   wo1_ref[...].astype(jnp.bfloat16),
                                   preferred_element_type=jnp.float32)
                           + bo1_ref[...]).astype(jnp.bfloat16),
                  wo2_ref[...].astype(jnp.bfloat16),
                  preferred_element_type=jnp.float32) + bo2_ref[...]
    out_ref[...] = out.reshape(1, TILE_N, F)


def _fused(rbf, gath, W1, b1, W2, b2, v_row, Wo1, bo1, Wo2, bo2):
    full = lambda shape: pl.BlockSpec(shape, lambda b, i: tuple(0 for _ in shape))
    return pl.pallas_call(
        _fused_body,
        grid=(B, NT),
        in_specs=[
            pl.BlockSpec((1, TILE_N, NBR, G), lambda b, i: (b, i, 0, 0)),
            pl.BlockSpec((1, TILE_N, NBR, F), lambda b, i: (b, i, 0, 0)),
            full((G, F)), full((1, F)), full((F, F)), full((1, F)),
            full((1, F)),
            full((F, F)), full((1, F)), full((F, F)), full((1, F)),
        ],
        out_specs=[
            pl.BlockSpec((1, TILE_N, F), lambda b, i: (b, i, 0)),
            pl.BlockSpec((1, TILE_N, NBR), lambda b, i: (b, i, 0)),
        ],
        out_shape=[
            jax.ShapeDtypeStruct((B, N, F), jnp.float32),
            jax.ShapeDtypeStruct((B, N, NBR), jnp.float32),
        ],
    )(rbf, gath, W1, b1, W2, b2, v_row, Wo1, bo1, Wo2, bo2)


def kernel(features, rbf_expansion, neighbor_list, W_init, W1, b1, W2, b2,
           nbr_filter, Wo1, bo1, Wo2, bo2):
    init = _init_features(features.reshape(B * N, F), W_init)
    b1r, b2r = b1.reshape(1, F), b2.reshape(1, F)
    bo1r, bo2r = bo1.reshape(1, F), bo2.reshape(1, F)
    v_row = nbr_filter.reshape(1, F)
    idx = (neighbor_list
           + (jnp.arange(B, dtype=jnp.int32) * N)[:, None, None])
    gath = _sc_gather(init, idx.reshape(1, B * N * NBR))
    out, attn = _fused(rbf_expansion, gath.reshape(B, N, NBR, F),
                       W1, b1r, W2, b2r, v_row, Wo1, bo1r, Wo2, bo2r)
    return out, attn


# speedup vs baseline: 1.9877x; 1.0741x over previous
import functools

import jax
import jax.numpy as jnp
from jax.experimental import pallas as pl
from jax.experimental.pallas import tpu as pltpu
from jax.experimental.pallas import tpu_sc as plsc

B, N, NBR = 4, 2500, 32
F, G = 128, 64

TILE_N = 256
NT = (N + TILE_N - 1) // TILE_N
GATHER_WINDOW = 128


def _init_body(feat_ref, w_ref, out_ref):
    out_ref[...] = jnp.dot(feat_ref[...], w_ref[...],
                           preferred_element_type=jnp.float32)


def _init_features(features_flat, W_init):
    return pl.pallas_call(
        _init_body,
        out_shape=jax.ShapeDtypeStruct((B * N, F), jnp.float32),
    )(features_flat, W_init)


def _sc_gather(table, idx_flat):
    m = idx_flat.shape[1]
    mesh = plsc.VectorSubcoreMesh(core_axis_name="c", subcore_axis_name="s")

    @functools.partial(
        pl.kernel,
        out_type=jax.ShapeDtypeStruct((m, F), jnp.float32),
        mesh=mesh,
    )
    def k(table_hbm, idx_hbm, out_hbm):
        def body(i_vmem, o_vmem):
            pltpu.sync_copy(table_hbm.at[i_vmem.at[0]], o_vmem)

        pltpu.emit_pipeline(
            body,
            grid=(m // GATHER_WINDOW,),
            in_specs=[pl.BlockSpec((1, GATHER_WINDOW), lambda i: (0, i))],
            out_specs=[pl.BlockSpec((GATHER_WINDOW, F), lambda i: (i, 0))],
            core_axis_name=("c", "s"),
            dimension_semantics=(pltpu.PARALLEL,),
        )(idx_hbm, out_hbm)

    return k(table, idx_flat)


def _fused_body(rbf_ref, gath_ref, w1_ref, b1_ref, w2_ref, b2_ref, v_ref,
                wo1_ref, bo1_ref, wo2_ref, bo2_ref, out_ref, attn_ref):
    rbf = rbf_ref[...].reshape(TILE_N * NBR, G).astype(jnp.bfloat16)
    h = jnp.tanh(jnp.dot(rbf, w1_ref[...].astype(jnp.bfloat16),
                         preferred_element_type=jnp.float32) + b1_ref[...])
    filt = jnp.dot(h.astype(jnp.bfloat16), w2_ref[...].astype(jnp.bfloat16),
                   preferred_element_type=jnp.float32) + b2_ref[...]
    conv = gath_ref[...].reshape(TILE_N * NBR, F) * filt
    conv3 = conv.reshape(TILE_N, NBR, F)
    logits = jnp.sum(conv3 * v_ref[...].reshape(1, 1, F), axis=-1,
                     keepdims=True)
    m = jnp.max(logits, axis=1, keepdims=True)
    e = jnp.exp(logits - m)
    attn3 = e / jnp.sum(e, axis=1, keepdims=True)
    attn_ref[...] = attn3.reshape(1, TILE_N, NBR)
    agg = jnp.sum(conv3 * attn3, axis=1)
    out = jnp.dot(jnp.tanh(jnp.dot(agg.astype(jnp.bfloat16),
                                   wo1_ref[...].astype(jnp.bfloat16),
                                   preferred_element_type=jnp.float32)
                           + bo1_ref[...]).astype(jnp.bfloat16),
                  wo2_ref[...].astype(jnp.bfloat16),
                  preferred_element_type=jnp.float32) + bo2_ref[...]
    out_ref[...] = out.reshape(1, TILE_N, F)


def _fused(rbf, gath, W1, b1, W2, b2, v_row, Wo1, bo1, Wo2, bo2):
    full = lambda shape: pl.BlockSpec(shape, lambda b, i: tuple(0 for _ in shape))
    return pl.pallas_call(
        _fused_body,
        grid=(B, NT),
        in_specs=[
            pl.BlockSpec((1, TILE_N, NBR, G), lambda b, i: (b, i, 0, 0)),
            pl.BlockSpec((1, TILE_N, NBR, F), lambda b, i: (b, i, 0, 0)),
            full((G, F)), full((1, F)), full((F, F)), full((1, F)),
            full((1, F)),
            full((F, F)), full((1, F)), full((F, F)), full((1, F)),
        ],
        out_specs=[
            pl.BlockSpec((1, TILE_N, F), lambda b, i: (b, i, 0)),
            pl.BlockSpec((1, TILE_N, NBR), lambda b, i: (b, i, 0)),
        ],
        out_shape=[
            jax.ShapeDtypeStruct((B, N, F), jnp.float32),
            jax.ShapeDtypeStruct((B, N, NBR), jnp.float32),
        ],
    )(rbf, gath, W1, b1, W2, b2, v_row, Wo1, bo1, Wo2, bo2)


def kernel(features, rbf_expansion, neighbor_list, W_init, W1, b1, W2, b2,
           nbr_filter, Wo1, bo1, Wo2, bo2):
    init = _init_features(features.reshape(B * N, F), W_init)
    b1r, b2r = b1.reshape(1, F), b2.reshape(1, F)
    bo1r, bo2r = bo1.reshape(1, F), bo2.reshape(1, F)
    v_row = nbr_filter.reshape(1, F)
    idx = (neighbor_list
           + (jnp.arange(B, dtype=jnp.int32) * N)[:, None, None])
    gath = _sc_gather(init, idx.reshape(1, B * N * NBR))
    out, attn = _fused(rbf_expansion, gath.reshape(B, N, NBR, F),
                       W1, b1r, W2, b2r, v_row, Wo1, bo1r, Wo2, bo2r)
    return out, attn


# speedup vs baseline: 2.0502x; 1.0314x over previous
import functools

import jax
import jax.numpy as jnp
from jax.experimental import pallas as pl
from jax.experimental.pallas import tpu as pltpu
from jax.experimental.pallas import tpu_sc as plsc

B, N, NBR = 4, 2500, 32
F, G = 128, 64

TILE_N = 512
NT = (N + TILE_N - 1) // TILE_N
GATHER_WINDOW = 128


def _init_body(feat_ref, w_ref, out_ref):
    out_ref[...] = jnp.dot(feat_ref[...], w_ref[...],
                           preferred_element_type=jnp.float32)


def _init_features(features_flat, W_init):
    return pl.pallas_call(
        _init_body,
        out_shape=jax.ShapeDtypeStruct((B * N, F), jnp.float32),
    )(features_flat, W_init)


def _sc_gather(table, idx_flat):
    m = idx_flat.shape[1]
    mesh = plsc.VectorSubcoreMesh(core_axis_name="c", subcore_axis_name="s")

    @functools.partial(
        pl.kernel,
        out_type=jax.ShapeDtypeStruct((m, F), jnp.float32),
        mesh=mesh,
    )
    def k(table_hbm, idx_hbm, out_hbm):
        def body(i_vmem, o_vmem):
            pltpu.sync_copy(table_hbm.at[i_vmem.at[0]], o_vmem)

        pltpu.emit_pipeline(
            body,
            grid=(m // GATHER_WINDOW,),
            in_specs=[pl.BlockSpec((1, GATHER_WINDOW), lambda i: (0, i))],
            out_specs=[pl.BlockSpec((GATHER_WINDOW, F), lambda i: (i, 0))],
            core_axis_name=("c", "s"),
            dimension_semantics=(pltpu.PARALLEL,),
        )(idx_hbm, out_hbm)

    return k(table, idx_flat)


def _fused_body(rbf_ref, gath_ref, w1_ref, b1_ref, w2_ref, b2_ref, v_ref,
                wo1_ref, bo1_ref, wo2_ref, bo2_ref, out_ref, attn_ref):
    rbf = rbf_ref[...].reshape(TILE_N * NBR, G).astype(jnp.bfloat16)
    h = jnp.tanh(jnp.dot(rbf, w1_ref[...].astype(jnp.bfloat16),
                         preferred_element_type=jnp.float32) + b1_ref[...])
    filt = jnp.dot(h.astype(jnp.bfloat16), w2_ref[...].astype(jnp.bfloat16),
                   preferred_element_type=jnp.float32) + b2_ref[...]
    conv = gath_ref[...].reshape(TILE_N * NBR, F) * filt
    conv3 = conv.reshape(TILE_N, NBR, F)
    logits = jnp.sum(conv3 * v_ref[...].reshape(1, 1, F), axis=-1,
                     keepdims=True)
    m = jnp.max(logits, axis=1, keepdims=True)
    e = jnp.exp(logits - m)
    attn3 = e / jnp.sum(e, axis=1, keepdims=True)
    attn_ref[...] = attn3.reshape(1, TILE_N, NBR)
    agg = jnp.sum(conv3 * attn3, axis=1)
    out = jnp.dot(jnp.tanh(jnp.dot(agg.astype(jnp.bfloat16),
                                   wo1_ref[...].astype(jnp.bfloat16),
                                   preferred_element_type=jnp.float32)
                           + bo1_ref[...]).astype(jnp.bfloat16),
                  wo2_ref[...].astype(jnp.bfloat16),
                  preferred_element_type=jnp.float32) + bo2_ref[...]
    out_ref[...] = out.reshape(1, TILE_N, F)


def _fused(rbf, gath, W1, b1, W2, b2, v_row, Wo1, bo1, Wo2, bo2):
    full = lambda shape: pl.BlockSpec(shape, lambda b, i: tuple(0 for _ in shape))
    return pl.pallas_call(
        _fused_body,
        grid=(B, NT),
        in_specs=[
            pl.BlockSpec((1, TILE_N, NBR, G), lambda b, i: (b, i, 0, 0)),
            pl.BlockSpec((1, TILE_N, NBR, F), lambda b, i: (b, i, 0, 0)),
            full((G, F)), full((1, F)), full((F, F)), full((1, F)),
            full((1, F)),
            full((F, F)), full((1, F)), full((F, F)), full((1, F)),
        ],
        out_specs=[
            pl.BlockSpec((1, TILE_N, F), lambda b, i: (b, i, 0)),
            pl.BlockSpec((1, TILE_N, NBR), lambda b, i: (b, i, 0)),
        ],
        out_shape=[
            jax.ShapeDtypeStruct((B, N, F), jnp.float32),
            jax.ShapeDtypeStruct((B, N, NBR), jnp.float32),
        ],
    )(rbf, gath, W1, b1, W2, b2, v_row, Wo1, bo1, Wo2, bo2)


def kernel(features, rbf_expansion, neighbor_list, W_init, W1, b1, W2, b2,
           nbr_filter, Wo1, bo1, Wo2, bo2):
    init = _init_features(features.reshape(B * N, F), W_init)
    b1r, b2r = b1.reshape(1, F), b2.reshape(1, F)
    bo1r, bo2r = bo1.reshape(1, F), bo2.reshape(1, F)
    v_row = nbr_filter.reshape(1, F)
    idx = (neighbor_list
           + (jnp.arange(B, dtype=jnp.int32) * N)[:, None, None])
    gath = _sc_gather(init, idx.reshape(1, B * N * NBR))
    out, attn = _fused(rbf_expansion, gath.reshape(B, N, NBR, F),
                       W1, b1r, W2, b2r, v_row, Wo1, bo1r, Wo2, bo2r)
    return out, attn
